# baseline (speedup 1.0000x reference)
import functools

import jax
import jax.numpy as jnp
from jax import lax
from jax.experimental import pallas as pl
from jax.experimental.pallas import tpu as pltpu
from jax.experimental.pallas import tpu_sc as plsc

MODEL_DIM = 1024
MAX_SEQ_LEN = 2048
BATCH = 32
NC, NS = 2, 16
NW = NC * NS
ROWS_W = MAX_SEQ_LEN // NW
WIN = 16
ZPOOL = 32
DEPTH = 2

_mesh = plsc.VectorSubcoreMesh(
    core_axis_name="c", subcore_axis_name="s", num_cores=NC, num_subcores=NS
)


@functools.partial(
    pl.kernel,
    out_type=jax.ShapeDtypeStruct((BATCH, MAX_SEQ_LEN, MODEL_DIM), jnp.float32),
    mesh=_mesh,
    scratch_types=[
        pltpu.VMEM((ROWS_W + ZPOOL, MODEL_DIM), jnp.float32),
        pltpu.VMEM((WIN, MODEL_DIM), jnp.float32),
        pltpu.VMEM((BATCH,), jnp.int32),
        pltpu.SemaphoreType.DMA,
        pltpu.SemaphoreType.DMA,
        pltpu.SemaphoreType.DMA,
    ],
    compiler_params=pltpu.CompilerParams(needs_layout_passes=False),
)
def _pe_lookup(
    len_hbm, table_hbm, shift_hbm, out_hbm, src_v, mix_v, len_v, sem_a, sem_g, sem_w
):
    w = lax.axis_index("s") * NC + lax.axis_index("c")
    w0 = w * ROWS_W
    lanes = lax.iota(jnp.int32, WIN)
    pltpu.make_async_copy(
        shift_hbm.at[pl.ds(w0, ROWS_W)], src_v.at[pl.ds(0, ROWS_W)], sem_g
    ).start()
    for g in range(ZPOOL // WIN):
        pltpu.make_async_copy(
            table_hbm.at[lanes * 0],
            src_v.at[pl.ds(ROWS_W + g * WIN, WIN)],
            sem_g,
        ).start()
    pltpu.sync_copy(len_hbm, len_v)
    pltpu.make_async_copy(
        shift_hbm.at[pl.ds(w0, ROWS_W)], src_v.at[pl.ds(0, ROWS_W)], sem_g
    ).wait()
    for g in range(ZPOOL // WIN):
        pltpu.make_async_copy(
            table_hbm.at[lanes], src_v.at[pl.ds(ROWS_W + g * WIN, WIN)], sem_g
        ).wait()
    lo = len_v[pl.ds(0, 16)]
    hi = len_v[pl.ds(16, 16)]

    def rows_of(b):
        in_lo = b < 16
        lane = jnp.where(in_lo, b, b - 16)
        vec = jnp.where(in_lo, lo, hi)
        len_b = jnp.sum(jnp.where(lanes == lane, vec, 0))
        return jnp.clip(len_b - w0, 0, ROWS_W)

    def linear_writes(b, r, run):
        a = (r // WIN) * WIN
        m = r - a
        for sz in (16, 32, 48, 64):

            @pl.when(a == sz)
            def _():
                run(src_v.at[pl.ds(0, sz)], out_hbm.at[b, pl.ds(w0, sz)])

        zs = a + jnp.where(m > 0, WIN, 0)
        z = ROWS_W - zs

        @pl.when(z >= 32)
        def _():
            run(src_v.at[pl.ds(ROWS_W, 32)], out_hbm.at[b, pl.ds(w0 + zs, 32)])

        @pl.when(z == 64)
        def _():
            run(
                src_v.at[pl.ds(ROWS_W, 32)],
                out_hbm.at[b, pl.ds(w0 + zs + 32, 32)],
            )

        @pl.when(z % 32 == WIN)
        def _():
            run(
                src_v.at[pl.ds(ROWS_W, WIN)],
                out_hbm.at[b, pl.ds(w0 + zs + z - WIN, WIN)],
            )

    def start(src, dst):
        pltpu.make_async_copy(src, dst, sem_a).start()

    def wait(src, dst):
        pltpu.make_async_copy(src, dst, sem_a).wait()

    def ragged_write_wait(b, r):
        a = (r // WIN) * WIN

        @pl.when(r - a > 0)
        def _():
            pltpu.make_async_copy(
                mix_v, out_hbm.at[b, pl.ds(w0 + a, WIN)], sem_w
            ).wait()

    def body(b, carry):
        rs = carry
        r = rows_of(b)
        a = (r // WIN) * WIN
        m = r - a

        ragged_write_wait(b - 1, rs[0])

        @pl.when(m > 0)
        def _():
            idx = jnp.where(a + lanes < r, w0 + a + lanes + 1, 0)
            pltpu.make_async_copy(table_hbm.at[idx], mix_v, sem_g).start()

        linear_writes(b, r, start)

        @pl.when(m > 0)
        def _():
            idx = jnp.where(a + lanes < r, w0 + a + lanes + 1, 0)
            pltpu.make_async_copy(table_hbm.at[idx], mix_v, sem_g).wait()
            pltpu.make_async_copy(
                mix_v, out_hbm.at[b, pl.ds(w0 + a, WIN)], sem_w
            ).start()

        @pl.when(b >= DEPTH)
        def _():
            linear_writes(b - DEPTH, rs[-1], wait)

        return (r,) + rs[:-1]

    rs = lax.fori_loop(0, BATCH, body, (jnp.int32(0),) * DEPTH)
    for d in range(DEPTH, 0, -1):
        linear_writes(BATCH - d, rs[d - 1], wait)
    ragged_write_wait(BATCH - 1, rs[0])


def kernel(input_len, pe_table):
    return _pe_lookup(input_len, pe_table, pe_table[1:])


# speedup vs baseline: 1.2718x; 1.2718x over previous
import functools

import jax
import jax.numpy as jnp
from jax import lax
from jax.experimental import pallas as pl
from jax.experimental.pallas import tpu as pltpu
from jax.experimental.pallas import tpu_sc as plsc

MODEL_DIM = 1024
MAX_SEQ_LEN = 2048
BATCH = 32
NC, NS = 2, 16
NW = NC * NS
ROWS_W = MAX_SEQ_LEN // NW
WIN = 16
ZPOOL = 32
DEPTH = 2

_mesh = plsc.VectorSubcoreMesh(
    core_axis_name="c", subcore_axis_name="s", num_cores=NC, num_subcores=NS
)


@functools.partial(
    pl.kernel,
    out_type=jax.ShapeDtypeStruct((BATCH, MAX_SEQ_LEN, MODEL_DIM), jnp.float32),
    mesh=_mesh,
    scratch_types=[
        pltpu.VMEM((ROWS_W + ZPOOL, MODEL_DIM), jnp.float32),
        pltpu.VMEM((WIN, MODEL_DIM), jnp.float32),
        pltpu.VMEM((BATCH,), jnp.int32),
        pltpu.SemaphoreType.DMA,
        pltpu.SemaphoreType.DMA,
        pltpu.SemaphoreType.DMA,
    ],
    compiler_params=pltpu.CompilerParams(needs_layout_passes=False),
)
def _pe_lookup(
    len_hbm, table_hbm, zero_hbm, out_hbm, src_v, mix_v, len_v, sem_a, sem_g, sem_w
):
    w = lax.axis_index("s") * NC + lax.axis_index("c")
    w0 = w * ROWS_W
    lanes = lax.iota(jnp.int32, WIN)
    for g in range(ROWS_W // WIN):
        pltpu.make_async_copy(
            table_hbm.at[w0 + 1 + g * WIN + lanes],
            src_v.at[pl.ds(g * WIN, WIN)],
            sem_g,
        ).start()
    pltpu.make_async_copy(
        zero_hbm, src_v.at[pl.ds(ROWS_W, ZPOOL)], sem_g
    ).start()
    pltpu.sync_copy(len_hbm, len_v)
    for g in range(ROWS_W // WIN):
        pltpu.make_async_copy(
            table_hbm.at[lanes], src_v.at[pl.ds(g * WIN, WIN)], sem_g
        ).wait()
    pltpu.make_async_copy(
        zero_hbm, src_v.at[pl.ds(ROWS_W, ZPOOL)], sem_g
    ).wait()
    lo = len_v[pl.ds(0, 16)]
    hi = len_v[pl.ds(16, 16)]

    def rows_of(b):
        in_lo = b < 16
        lane = jnp.where(in_lo, b, b - 16)
        vec = jnp.where(in_lo, lo, hi)
        len_b = jnp.sum(jnp.where(lanes == lane, vec, 0))
        return jnp.clip(len_b - w0, 0, ROWS_W)

    def linear_writes(b, r, run):
        a = (r // WIN) * WIN
        m = r - a
        for sz in (16, 32, 48, 64):

            @pl.when(a == sz)
            def _():
                run(src_v.at[pl.ds(0, sz)], out_hbm.at[b, pl.ds(w0, sz)])

        zs = a + jnp.where(m > 0, WIN, 0)
        z = ROWS_W - zs

        @pl.when(z >= 32)
        def _():
            run(src_v.at[pl.ds(ROWS_W, 32)], out_hbm.at[b, pl.ds(w0 + zs, 32)])

        @pl.when(z == 64)
        def _():
            run(
                src_v.at[pl.ds(ROWS_W, 32)],
                out_hbm.at[b, pl.ds(w0 + zs + 32, 32)],
            )

        @pl.when(z % 32 == WIN)
        def _():
            run(
                src_v.at[pl.ds(ROWS_W, WIN)],
                out_hbm.at[b, pl.ds(w0 + zs + z - WIN, WIN)],
            )

    def start(src, dst):
        pltpu.make_async_copy(src, dst, sem_a).start()

    def wait(src, dst):
        pltpu.make_async_copy(src, dst, sem_a).wait()

    def ragged_write_wait(b, r):
        a = (r // WIN) * WIN

        @pl.when(r - a > 0)
        def _():
            pltpu.make_async_copy(
                mix_v, out_hbm.at[b, pl.ds(w0 + a, WIN)], sem_w
            ).wait()

    def body(b, carry):
        rs = carry
        r = rows_of(b)
        a = (r // WIN) * WIN
        m = r - a

        ragged_write_wait(b - 1, rs[0])

        @pl.when(m > 0)
        def _():
            idx = jnp.where(a + lanes < r, w0 + a + lanes + 1, 0)
            pltpu.make_async_copy(table_hbm.at[idx], mix_v, sem_g).start()

        linear_writes(b, r, start)

        @pl.when(m > 0)
        def _():
            idx = jnp.where(a + lanes < r, w0 + a + lanes + 1, 0)
            pltpu.make_async_copy(table_hbm.at[idx], mix_v, sem_g).wait()
            pltpu.make_async_copy(
                mix_v, out_hbm.at[b, pl.ds(w0 + a, WIN)], sem_w
            ).start()

        @pl.when(b >= DEPTH)
        def _():
            linear_writes(b - DEPTH, rs[-1], wait)

        return (r,) + rs[:-1]

    rs = lax.fori_loop(0, BATCH, body, (jnp.int32(0),) * DEPTH)
    for d in range(DEPTH, 0, -1):
        linear_writes(BATCH - d, rs[d - 1], wait)
    ragged_write_wait(BATCH - 1, rs[0])


def kernel(input_len, pe_table):
    zeros = jnp.zeros((ZPOOL, MODEL_DIM), jnp.float32)
    return _pe_lookup(input_len, pe_table, zeros)


# speedup vs baseline: 1.2756x; 1.0030x over previous
import functools

import jax
import jax.numpy as jnp
from jax import lax
from jax.experimental import pallas as pl
from jax.experimental.pallas import tpu as pltpu
from jax.experimental.pallas import tpu_sc as plsc

MODEL_DIM = 1024
MAX_SEQ_LEN = 2048
BATCH = 32
NC, NS = 2, 16
NW = NC * NS
ROWS_W = MAX_SEQ_LEN // NW
WIN = 16
ZPOOL = 32
DEPTH = 3

_mesh = plsc.VectorSubcoreMesh(
    core_axis_name="c", subcore_axis_name="s", num_cores=NC, num_subcores=NS
)


@functools.partial(
    pl.kernel,
    out_type=jax.ShapeDtypeStruct((BATCH, MAX_SEQ_LEN, MODEL_DIM), jnp.float32),
    mesh=_mesh,
    scratch_types=[
        pltpu.VMEM((ROWS_W + ZPOOL, MODEL_DIM), jnp.float32),
        pltpu.VMEM((WIN, MODEL_DIM), jnp.float32),
        pltpu.VMEM((BATCH,), jnp.int32),
        pltpu.SemaphoreType.DMA,
        pltpu.SemaphoreType.DMA,
        pltpu.SemaphoreType.DMA,
    ],
    compiler_params=pltpu.CompilerParams(needs_layout_passes=False),
)
def _pe_lookup(
    len_hbm, table_hbm, zero_hbm, out_hbm, src_v, mix_v, len_v, sem_a, sem_g, sem_w
):
    w = lax.axis_index("s") * NC + lax.axis_index("c")
    w0 = w * ROWS_W
    lanes = lax.iota(jnp.int32, WIN)
    for g in range(ROWS_W // WIN):
        pltpu.make_async_copy(
            table_hbm.at[w0 + 1 + g * WIN + lanes],
            src_v.at[pl.ds(g * WIN, WIN)],
            sem_g,
        ).start()
    pltpu.make_async_copy(
        zero_hbm, src_v.at[pl.ds(ROWS_W, ZPOOL)], sem_g
    ).start()
    pltpu.sync_copy(len_hbm, len_v)
    for g in range(ROWS_W // WIN):
        pltpu.make_async_copy(
            table_hbm.at[lanes], src_v.at[pl.ds(g * WIN, WIN)], sem_g
        ).wait()
    pltpu.make_async_copy(
        zero_hbm, src_v.at[pl.ds(ROWS_W, ZPOOL)], sem_g
    ).wait()
    lo = len_v[pl.ds(0, 16)]
    hi = len_v[pl.ds(16, 16)]

    def rows_of(b):
        in_lo = b < 16
        lane = jnp.where(in_lo, b, b - 16)
        vec = jnp.where(in_lo, lo, hi)
        len_b = jnp.sum(jnp.where(lanes == lane, vec, 0))
        return jnp.clip(len_b - w0, 0, ROWS_W)

    def linear_writes(b, r, run):
        a = (r // WIN) * WIN
        m = r - a
        for sz in (16, 32, 48, 64):

            @pl.when(a == sz)
            def _():
                run(src_v.at[pl.ds(0, sz)], out_hbm.at[b, pl.ds(w0, sz)])

        zs = a + jnp.where(m > 0, WIN, 0)
        z = ROWS_W - zs

        @pl.when(z >= 32)
        def _():
            run(src_v.at[pl.ds(ROWS_W, 32)], out_hbm.at[b, pl.ds(w0 + zs, 32)])

        @pl.when(z == 64)
        def _():
            run(
                src_v.at[pl.ds(ROWS_W, 32)],
                out_hbm.at[b, pl.ds(w0 + zs + 32, 32)],
            )

        @pl.when(z % 32 == WIN)
        def _():
            run(
                src_v.at[pl.ds(ROWS_W, WIN)],
                out_hbm.at[b, pl.ds(w0 + zs + z - WIN, WIN)],
            )

    def start(src, dst):
        pltpu.make_async_copy(src, dst, sem_a).start()

    def wait(src, dst):
        pltpu.make_async_copy(src, dst, sem_a).wait()

    def ragged_write_wait(b, r):
        a = (r // WIN) * WIN

        @pl.when(r - a > 0)
        def _():
            pltpu.make_async_copy(
                mix_v, out_hbm.at[b, pl.ds(w0 + a, WIN)], sem_w
            ).wait()

    def body(b, carry):
        rs = carry
        r = rows_of(b)
        a = (r // WIN) * WIN
        m = r - a

        ragged_write_wait(b - 1, rs[0])

        @pl.when(m > 0)
        def _():
            idx = jnp.where(a + lanes < r, w0 + a + lanes + 1, 0)
            pltpu.make_async_copy(table_hbm.at[idx], mix_v, sem_g).start()

        linear_writes(b, r, start)

        @pl.when(m > 0)
        def _():
            idx = jnp.where(a + lanes < r, w0 + a + lanes + 1, 0)
            pltpu.make_async_copy(table_hbm.at[idx], mix_v, sem_g).wait()
            pltpu.make_async_copy(
                mix_v, out_hbm.at[b, pl.ds(w0 + a, WIN)], sem_w
            ).start()

        @pl.when(b >= DEPTH)
        def _():
            linear_writes(b - DEPTH, rs[-1], wait)

        return (r,) + rs[:-1]

    rs = lax.fori_loop(0, BATCH, body, (jnp.int32(0),) * DEPTH)
    for d in range(DEPTH, 0, -1):
        linear_writes(BATCH - d, rs[d - 1], wait)
    ragged_write_wait(BATCH - 1, rs[0])


def kernel(input_len, pe_table):
    zeros = jnp.zeros((ZPOOL, MODEL_DIM), jnp.float32)
    return _pe_lookup(input_len, pe_table, zeros)
